# baseline (speedup 1.0000x reference)
import functools

import jax
import jax.numpy as jnp
from jax import lax
from jax.experimental import pallas as pl
from jax.experimental.pallas import tpu as pltpu
from jax.experimental.pallas import tpu_sc as plsc

NC = 2
NS = 16
NW = NC * NS
K = 128
ZR = 8
RB = 4
PF = 2


def _make_seg_sum(n_tab, d, cpws, n_pad, with_deg):
    ncc = len(cpws)
    cpw0 = cpws[0]
    cpw1 = cpws[1] if ncc > 1 else cpws[0]
    cpw_max = max(cpws)
    rpt = n_pad // NS
    mesh = plsc.VectorSubcoreMesh(core_axis_name="c", subcore_axis_name="s",
                                  num_cores=ncc)

    out_type = [jax.ShapeDtypeStruct((ncc, n_pad, d), jnp.float32)]
    scratch = [
        pltpu.VMEM_SHARED((n_pad, d), jnp.float32),
        pltpu.VMEM((cpw_max, K), jnp.int32),
        pltpu.VMEM((cpw_max, K), jnp.int32),
        [pltpu.VMEM((K, d), jnp.float32)] * RB,
        pltpu.VMEM((ZR, d), jnp.float32),
        [pltpu.SemaphoreType.DMA] * RB,
        [pltpu.SemaphoreType.DMA] * RB,
        pltpu.SemaphoreType.DMA,
    ]
    if with_deg:
        out_type.append(jax.ShapeDtypeStruct((ncc, n_pad, 16), jnp.float32))
        scratch += [
            pltpu.VMEM_SHARED((n_pad, 16), jnp.float32),
            pltpu.VMEM((K, 16), jnp.float32),
            pltpu.VMEM((ZR, 16), jnp.float32),
        ]

    def body(tab_hbm, src_hbm, dst_hbm, acc_out, *rest):
        if with_deg:
            (deg_out, acc_sh, idx_s, idx_d, rows, zbuf, sem_g, sem_s, sem_z,
             deg_sh, ones_b, z16) = rest
        else:
            (acc_sh, idx_s, idx_d, rows, zbuf, sem_g, sem_s, sem_z) = rest
        c = lax.axis_index("c")
        s = lax.axis_index("s")
        if ncc == 1:
            cpw = cpw0
            base_row = s * cpw0
        else:
            cpw = jnp.where(c == 0, cpw0, cpw1)
            base_row = jnp.where(c == 0, s * cpw0, NS * cpw0 + s * cpw1)
        zv = jnp.zeros((16,), jnp.float32)

        def start_gather(cur, b):
            pltpu.async_copy(tab_hbm.at[idx_s.at[cur]], rows[b], sem_g[b])

        def wait_gather(cur, b):
            pltpu.make_async_copy(tab_hbm.at[idx_s.at[cur]], rows[b],
                                  sem_g[b]).wait()

        def start_scatter(cur, b):
            pltpu.async_copy(rows[b], acc_sh.at[idx_d.at[cur]], sem_s[b],
                             add=True)
            if with_deg:
                pltpu.async_copy(ones_b, deg_sh.at[idx_d.at[cur]], sem_s[b],
                                 add=True)

        def wait_scatter(cur, b):
            pltpu.make_async_copy(rows[b], acc_sh.at[idx_d.at[cur]],
                                  sem_s[b]).wait()
            if with_deg:
                pltpu.make_async_copy(ones_b, deg_sh.at[idx_d.at[cur]],
                                      sem_s[b]).wait()

        cpw_min = min(cpws)
        pltpu.sync_copy(src_hbm.at[pl.ds(base_row, cpw_min)],
                        idx_s.at[pl.ds(0, cpw_min)])
        pltpu.sync_copy(dst_hbm.at[pl.ds(base_row, cpw_min)],
                        idx_d.at[pl.ds(0, cpw_min)])
        if cpw_max > cpw_min:
            big = 0 if cpw0 >= cpw1 else 1

            @pl.when(c == big)
            def _():
                pltpu.sync_copy(
                    src_hbm.at[pl.ds(base_row + cpw_min, cpw_max - cpw_min)],
                    idx_s.at[pl.ds(cpw_min, cpw_max - cpw_min)])
                pltpu.sync_copy(
                    dst_hbm.at[pl.ds(base_row + cpw_min, cpw_max - cpw_min)],
                    idx_d.at[pl.ds(cpw_min, cpw_max - cpw_min)])
        for b in range(PF):
            start_gather(b, b)

        @pl.loop(0, ZR)
        def _(r):
            for j in range(d // 16):
                zbuf[r, pl.ds(j * 16, 16)] = zv
            if with_deg:
                z16[r, :] = zv

        if with_deg:
            @pl.loop(0, K)
            def _(r):
                ones_b[r, :] = jnp.full((16,), 1.0, jnp.float32)

        @pl.loop(0, rpt // ZR)
        def _(t):
            r0 = s * rpt + t * ZR
            pltpu.async_copy(zbuf, acc_sh.at[pl.ds(r0, ZR)], sem_z)
            if with_deg:
                pltpu.async_copy(z16, deg_sh.at[pl.ds(r0, ZR)], sem_z)

        @pl.loop(0, rpt // ZR)
        def _(t):
            r0 = s * rpt + t * ZR
            pltpu.make_async_copy(zbuf, acc_sh.at[pl.ds(r0, ZR)],
                                  sem_z).wait()
            if with_deg:
                pltpu.make_async_copy(z16, deg_sh.at[pl.ds(r0, ZR)],
                                      sem_z).wait()

        plsc.subcore_barrier()

        @pl.loop(0, cpw, step=RB)
        def _(i):
            for b in range(RB):
                cur = i + b
                wait_gather(cur, b)
                start_scatter(cur, b)
                nxt = cur + PF
                bn = (b + PF) % RB
                prev = nxt - RB

                @pl.when(nxt < cpw)
                def _():
                    @pl.when(prev >= 0)
                    def _():
                        wait_scatter(prev, bn)
                    start_gather(nxt, bn)

        for k in range(RB):
            wait_scatter(cpw - RB + k, k)

        plsc.subcore_barrier()
        r0 = s * rpt
        pltpu.sync_copy(acc_sh.at[pl.ds(r0, rpt)],
                        acc_out.at[c, pl.ds(r0, rpt)])
        if with_deg:
            pltpu.sync_copy(deg_sh.at[pl.ds(r0, rpt)],
                            deg_out.at[c, pl.ds(r0, rpt)])

    return pl.kernel(body, out_type=tuple(out_type) if with_deg else out_type[0],
                     mesh=mesh, scratch_types=scratch,
                     compiler_params=pltpu.CompilerParams(
                         use_tc_tiling_on_sc=False))


def _tc_pre(x_ref, w1r_ref, b1_ref, r1_ref):
    r1_ref[...] = (jnp.dot(x_ref[...], w1r_ref[...],
                           preferred_element_type=jnp.float32) + b1_ref[...])


def _tc_mid(acca_ref, accb_ref, deg_ref, r1_ref, w1l_ref,
            w2l_ref, w2r_ref, b2_ref, p2_ref, base2_ref):
    dh = acca_ref.shape[-1]
    ncc = acca_ref.shape[0]
    deg = sum(deg_ref[i, :, 0] for i in range(ncc))
    invd = 1.0 / jnp.maximum(deg, 1.0)
    agg_a = sum(acca_ref[i] for i in range(ncc)) * invd[:, None]
    agg_b = sum(accb_ref[i] for i in range(ncc)) * invd[:, None]
    h = (jnp.dot(agg_a, w1l_ref[:dh], preferred_element_type=jnp.float32)
         + jnp.dot(agg_b, w1l_ref[dh:], preferred_element_type=jnp.float32)
         + r1_ref[...])
    h = jnp.maximum(h, 0.0)
    p2_ref[...] = jnp.dot(h, w2l_ref[...], preferred_element_type=jnp.float32)
    base2_ref[...] = (jnp.dot(h, w2r_ref[...], preferred_element_type=jnp.float32)
                      + b2_ref[...])


def _tc_out(acc_ref, deg_ref, base2_ref, out_ref):
    ncc = acc_ref.shape[0]
    deg = sum(deg_ref[i, :, 0] for i in range(ncc))
    invd = 1.0 / jnp.maximum(deg, 1.0)
    c = out_ref.shape[-1]
    agg = sum(acc_ref[i, :, :c] for i in range(ncc)) * invd[:, None]
    out_ref[...] = agg + base2_ref[...]


def kernel(x, edge_index, W1_l, b1, W1_r, W2_l, b2, W2_r):
    n, d = x.shape
    dh = d // 2
    h_dim = W1_l.shape[1]
    c_dim = W2_l.shape[1]
    c_pad = 48
    e = edge_index.shape[1]

    blk = 256
    n_blocks = pl.cdiv(n, blk)
    n_pad = pl.cdiv(n + 1, NS * 8) * NS * 8
    total_cpt = pl.cdiv(e, NS * K * 8) * 8
    cpw1 = max(8, (total_cpt // 160) * 8)
    cpws = (total_cpt - cpw1, cpw1)
    ncc = len(cpws)
    n_chunk_rows = NS * total_cpt
    e_pad = n_chunk_rows * K

    src = edge_index[0]
    dst = edge_index[1]
    pad = e_pad - e
    src2d = jnp.concatenate([src, jnp.zeros((pad,), jnp.int32)]).reshape(
        n_chunk_rows, K)
    dst2d = jnp.concatenate([dst, jnp.full((pad,), n, jnp.int32)]).reshape(
        n_chunk_rows, K)

    seg1 = _make_seg_sum(n, dh, cpws, n_pad, with_deg=True)
    x_a = x[:, :dh]
    x_b = x[:, dh:]
    acc_a, deg = seg1(x_a, src2d, dst2d)
    acc_b, _ = seg1(x_b, src2d, dst2d)

    w2l_p = jnp.pad(W2_l, ((0, 0), (0, c_pad - c_dim)))
    grid = (n_blocks,)
    r1 = pl.pallas_call(
        _tc_pre,
        grid=grid,
        in_specs=[
            pl.BlockSpec((blk, d), lambda i: (i, 0)),
            pl.BlockSpec((d, h_dim), lambda i: (0, 0)),
            pl.BlockSpec((1, h_dim), lambda i: (0, 0)),
        ],
        out_specs=pl.BlockSpec((blk, h_dim), lambda i: (i, 0)),
        out_shape=jax.ShapeDtypeStruct((n, h_dim), jnp.float32),
    )(x, W1_r, b1.reshape(1, h_dim))
    p2, base2 = pl.pallas_call(
        _tc_mid,
        grid=grid,
        in_specs=[
            pl.BlockSpec((ncc, blk, dh), lambda i: (0, i, 0)),
            pl.BlockSpec((ncc, blk, dh), lambda i: (0, i, 0)),
            pl.BlockSpec((ncc, blk, 16), lambda i: (0, i, 0)),
            pl.BlockSpec((blk, h_dim), lambda i: (i, 0)),
            pl.BlockSpec((d, h_dim), lambda i: (0, 0)),
            pl.BlockSpec((h_dim, c_pad), lambda i: (0, 0)),
            pl.BlockSpec((h_dim, c_dim), lambda i: (0, 0)),
            pl.BlockSpec((1, c_dim), lambda i: (0, 0)),
        ],
        out_specs=[
            pl.BlockSpec((blk, c_pad), lambda i: (i, 0)),
            pl.BlockSpec((blk, c_dim), lambda i: (i, 0)),
        ],
        out_shape=[
            jax.ShapeDtypeStruct((n, c_pad), jnp.float32),
            jax.ShapeDtypeStruct((n, c_dim), jnp.float32),
        ],
    )(acc_a, acc_b, deg, r1, W1_l, w2l_p, W2_r, b2.reshape(1, c_dim))

    seg2 = _make_seg_sum(n, c_pad, cpws, n_pad, with_deg=False)
    acc2 = seg2(p2, src2d, dst2d)

    out = pl.pallas_call(
        _tc_out,
        grid=grid,
        in_specs=[
            pl.BlockSpec((ncc, blk, c_pad), lambda i: (0, i, 0)),
            pl.BlockSpec((ncc, blk, 16), lambda i: (0, i, 0)),
            pl.BlockSpec((blk, c_dim), lambda i: (i, 0)),
        ],
        out_specs=pl.BlockSpec((blk, c_dim), lambda i: (i, 0)),
        out_shape=jax.ShapeDtypeStruct((n, c_dim), jnp.float32),
    )(acc2, deg, base2)
    return out


# speedup vs baseline: 1.0066x; 1.0066x over previous
import functools

import jax
import jax.numpy as jnp
from jax import lax
from jax.experimental import pallas as pl
from jax.experimental.pallas import tpu as pltpu
from jax.experimental.pallas import tpu_sc as plsc

NC = 2
NS = 16
NW = NC * NS
K = 128
ZR = 8
RB = 4
PF = 2


def _make_seg_sum(n_tab, d, cpws, n_pad, with_deg):
    ncc = len(cpws)
    cpw0 = cpws[0]
    cpw1 = cpws[1] if ncc > 1 else cpws[0]
    cpw_max = max(cpws)
    rpt = n_pad // NS
    mesh = plsc.VectorSubcoreMesh(core_axis_name="c", subcore_axis_name="s",
                                  num_cores=ncc)

    out_type = [jax.ShapeDtypeStruct((ncc, n_pad, d), jnp.float32)]
    scratch = [
        pltpu.VMEM_SHARED((n_pad, d), jnp.float32),
        pltpu.VMEM((cpw_max, K), jnp.int32),
        pltpu.VMEM((cpw_max, K), jnp.int32),
        [pltpu.VMEM((K, d), jnp.float32)] * RB,
        pltpu.VMEM((ZR, d), jnp.float32),
        [pltpu.SemaphoreType.DMA] * RB,
        [pltpu.SemaphoreType.DMA] * RB,
        pltpu.SemaphoreType.DMA,
    ]
    if with_deg:
        out_type.append(jax.ShapeDtypeStruct((ncc, n_pad, 16), jnp.float32))
        scratch += [
            pltpu.VMEM_SHARED((n_pad, 16), jnp.float32),
            pltpu.VMEM((K, 16), jnp.float32),
            pltpu.VMEM((ZR, 16), jnp.float32),
        ]

    def body(tab_hbm, src_hbm, dst_hbm, acc_out, *rest):
        if with_deg:
            (deg_out, acc_sh, idx_s, idx_d, rows, zbuf, sem_g, sem_s, sem_z,
             deg_sh, ones_b, z16) = rest
        else:
            (acc_sh, idx_s, idx_d, rows, zbuf, sem_g, sem_s, sem_z) = rest
        c = lax.axis_index("c")
        s = lax.axis_index("s")
        if ncc == 1:
            cpw = cpw0
            base_row = s * cpw0
        else:
            cpw = jnp.where(c == 0, cpw0, cpw1)
            base_row = jnp.where(c == 0, s * cpw0, NS * cpw0 + s * cpw1)
        zv = jnp.zeros((16,), jnp.float32)

        def start_gather(cur, b):
            pltpu.async_copy(tab_hbm.at[idx_s.at[cur]], rows[b], sem_g[b])

        def wait_gather(cur, b):
            pltpu.make_async_copy(tab_hbm.at[idx_s.at[cur]], rows[b],
                                  sem_g[b]).wait()

        def start_scatter(cur, b):
            pltpu.async_copy(rows[b], acc_sh.at[idx_d.at[cur]], sem_s[b],
                             add=True)
            if with_deg:
                pltpu.async_copy(ones_b, deg_sh.at[idx_d.at[cur]], sem_s[b],
                                 add=True)

        def wait_scatter(cur, b):
            pltpu.make_async_copy(rows[b], acc_sh.at[idx_d.at[cur]],
                                  sem_s[b]).wait()
            if with_deg:
                pltpu.make_async_copy(ones_b, deg_sh.at[idx_d.at[cur]],
                                      sem_s[b]).wait()

        cpw_min = min(cpws)
        pltpu.sync_copy(src_hbm.at[pl.ds(base_row, cpw_min)],
                        idx_s.at[pl.ds(0, cpw_min)])
        pltpu.sync_copy(dst_hbm.at[pl.ds(base_row, cpw_min)],
                        idx_d.at[pl.ds(0, cpw_min)])
        if cpw_max > cpw_min:
            big = 0 if cpw0 >= cpw1 else 1

            @pl.when(c == big)
            def _():
                pltpu.sync_copy(
                    src_hbm.at[pl.ds(base_row + cpw_min, cpw_max - cpw_min)],
                    idx_s.at[pl.ds(cpw_min, cpw_max - cpw_min)])
                pltpu.sync_copy(
                    dst_hbm.at[pl.ds(base_row + cpw_min, cpw_max - cpw_min)],
                    idx_d.at[pl.ds(cpw_min, cpw_max - cpw_min)])
        for b in range(PF):
            start_gather(b, b)

        @pl.loop(0, ZR)
        def _(r):
            for j in range(d // 16):
                zbuf[r, pl.ds(j * 16, 16)] = zv
            if with_deg:
                z16[r, :] = zv

        if with_deg:
            @pl.loop(0, K)
            def _(r):
                ones_b[r, :] = jnp.full((16,), 1.0, jnp.float32)

        @pl.loop(0, rpt // ZR)
        def _(t):
            r0 = s * rpt + t * ZR
            pltpu.async_copy(zbuf, acc_sh.at[pl.ds(r0, ZR)], sem_z)
            if with_deg:
                pltpu.async_copy(z16, deg_sh.at[pl.ds(r0, ZR)], sem_z)

        @pl.loop(0, rpt // ZR)
        def _(t):
            r0 = s * rpt + t * ZR
            pltpu.make_async_copy(zbuf, acc_sh.at[pl.ds(r0, ZR)],
                                  sem_z).wait()
            if with_deg:
                pltpu.make_async_copy(z16, deg_sh.at[pl.ds(r0, ZR)],
                                      sem_z).wait()

        plsc.subcore_barrier()

        @pl.loop(0, cpw, step=RB)
        def _(i):
            for b in range(RB):
                cur = i + b
                wait_gather(cur, b)
                start_scatter(cur, b)
                nxt = cur + PF
                bn = (b + PF) % RB
                prev = nxt - RB

                @pl.when(nxt < cpw)
                def _():
                    @pl.when(prev >= 0)
                    def _():
                        wait_scatter(prev, bn)
                    start_gather(nxt, bn)

        for k in range(RB):
            wait_scatter(cpw - RB + k, k)

        plsc.subcore_barrier()
        r0 = s * rpt
        pltpu.sync_copy(acc_sh.at[pl.ds(r0, rpt)],
                        acc_out.at[c, pl.ds(r0, rpt)])
        if with_deg:
            pltpu.sync_copy(deg_sh.at[pl.ds(r0, rpt)],
                            deg_out.at[c, pl.ds(r0, rpt)])

    return pl.kernel(body, out_type=tuple(out_type) if with_deg else out_type[0],
                     mesh=mesh, scratch_types=scratch,
                     compiler_params=pltpu.CompilerParams(
                         use_tc_tiling_on_sc=False))


def _tc_pre(x_ref, w1r_ref, b1_ref, r1_ref):
    r1_ref[...] = (jnp.dot(x_ref[...], w1r_ref[...],
                           preferred_element_type=jnp.float32) + b1_ref[...])


def _tc_mid(acca_ref, accb_ref, deg_ref, r1_ref, w1l_ref,
            w2l_ref, w2r_ref, b2_ref, p2_ref, base2_ref):
    dh = acca_ref.shape[-1]
    ncc = acca_ref.shape[0]
    deg = sum(deg_ref[i, :, 0] for i in range(ncc))
    invd = 1.0 / jnp.maximum(deg, 1.0)
    agg_a = sum(acca_ref[i] for i in range(ncc)) * invd[:, None]
    agg_b = sum(accb_ref[i] for i in range(ncc)) * invd[:, None]
    h = (jnp.dot(agg_a, w1l_ref[:dh], preferred_element_type=jnp.float32)
         + jnp.dot(agg_b, w1l_ref[dh:], preferred_element_type=jnp.float32)
         + r1_ref[...])
    h = jnp.maximum(h, 0.0)
    p2_ref[...] = jnp.dot(h, w2l_ref[...], preferred_element_type=jnp.float32)
    base2_ref[...] = (jnp.dot(h, w2r_ref[...], preferred_element_type=jnp.float32)
                      + b2_ref[...])


def _tc_out(acc_ref, deg_ref, base2_ref, out_ref):
    ncc = acc_ref.shape[0]
    deg = sum(deg_ref[i, :, 0] for i in range(ncc))
    invd = 1.0 / jnp.maximum(deg, 1.0)
    c = out_ref.shape[-1]
    agg = sum(acc_ref[i, :, :c] for i in range(ncc)) * invd[:, None]
    out_ref[...] = agg + base2_ref[...]


def kernel(x, edge_index, W1_l, b1, W1_r, W2_l, b2, W2_r):
    n, d = x.shape
    dh = d // 2
    h_dim = W1_l.shape[1]
    c_dim = W2_l.shape[1]
    c_pad = 48
    e = edge_index.shape[1]

    blk = 256
    n_blocks = pl.cdiv(n, blk)
    n_pad = pl.cdiv(n + 1, NS * 8) * NS * 8
    total_cpt = pl.cdiv(e, NS * K * 8) * 8
    cpw1 = max(8, (total_cpt // 80) * 8)
    cpws = (total_cpt - cpw1, cpw1)
    ncc = len(cpws)
    n_chunk_rows = NS * total_cpt
    e_pad = n_chunk_rows * K

    src = edge_index[0]
    dst = edge_index[1]
    pad = e_pad - e
    src2d = jnp.concatenate([src, jnp.zeros((pad,), jnp.int32)]).reshape(
        n_chunk_rows, K)
    dst2d = jnp.concatenate([dst, jnp.full((pad,), n, jnp.int32)]).reshape(
        n_chunk_rows, K)

    seg1 = _make_seg_sum(n, dh, cpws, n_pad, with_deg=True)
    x_a = x[:, :dh]
    x_b = x[:, dh:]
    acc_a, deg = seg1(x_a, src2d, dst2d)
    acc_b, _ = seg1(x_b, src2d, dst2d)

    w2l_p = jnp.pad(W2_l, ((0, 0), (0, c_pad - c_dim)))
    grid = (n_blocks,)
    r1 = pl.pallas_call(
        _tc_pre,
        grid=grid,
        in_specs=[
            pl.BlockSpec((blk, d), lambda i: (i, 0)),
            pl.BlockSpec((d, h_dim), lambda i: (0, 0)),
            pl.BlockSpec((1, h_dim), lambda i: (0, 0)),
        ],
        out_specs=pl.BlockSpec((blk, h_dim), lambda i: (i, 0)),
        out_shape=jax.ShapeDtypeStruct((n, h_dim), jnp.float32),
    )(x, W1_r, b1.reshape(1, h_dim))
    p2, base2 = pl.pallas_call(
        _tc_mid,
        grid=grid,
        in_specs=[
            pl.BlockSpec((ncc, blk, dh), lambda i: (0, i, 0)),
            pl.BlockSpec((ncc, blk, dh), lambda i: (0, i, 0)),
            pl.BlockSpec((ncc, blk, 16), lambda i: (0, i, 0)),
            pl.BlockSpec((blk, h_dim), lambda i: (i, 0)),
            pl.BlockSpec((d, h_dim), lambda i: (0, 0)),
            pl.BlockSpec((h_dim, c_pad), lambda i: (0, 0)),
            pl.BlockSpec((h_dim, c_dim), lambda i: (0, 0)),
            pl.BlockSpec((1, c_dim), lambda i: (0, 0)),
        ],
        out_specs=[
            pl.BlockSpec((blk, c_pad), lambda i: (i, 0)),
            pl.BlockSpec((blk, c_dim), lambda i: (i, 0)),
        ],
        out_shape=[
            jax.ShapeDtypeStruct((n, c_pad), jnp.float32),
            jax.ShapeDtypeStruct((n, c_dim), jnp.float32),
        ],
    )(acc_a, acc_b, deg, r1, W1_l, w2l_p, W2_r, b2.reshape(1, c_dim))

    seg2 = _make_seg_sum(n, c_pad, cpws, n_pad, with_deg=False)
    acc2 = seg2(p2, src2d, dst2d)

    out = pl.pallas_call(
        _tc_out,
        grid=grid,
        in_specs=[
            pl.BlockSpec((ncc, blk, c_pad), lambda i: (0, i, 0)),
            pl.BlockSpec((ncc, blk, 16), lambda i: (0, i, 0)),
            pl.BlockSpec((blk, c_dim), lambda i: (i, 0)),
        ],
        out_specs=pl.BlockSpec((blk, c_dim), lambda i: (i, 0)),
        out_shape=jax.ShapeDtypeStruct((n, c_dim), jnp.float32),
    )(acc2, deg, base2)
    return out
